# Initial kernel scaffold; baseline (speedup 1.0000x reference)
import jax
import jax.numpy as jnp
from jax.experimental import pallas as pl


def kernel(node_memory, last_update, edge_index, event_t, event_msg, wt, bt, Wq, bq, Wk, bk, Wv, bv, We, be, Ws, bs):
    raise NotImplementedError("write your pallas kernel here")


# speedup vs baseline: 12.7297x; 12.7297x over previous
import functools

import jax
import jax.numpy as jnp
from jax import lax
from jax.experimental import pallas as pl
from jax.experimental.pallas import tpu as pltpu
from jax.experimental.pallas import tpu_sc as plsc

N = 10000
E = 320000
D = 128
H = 2
C = 64
TD = 32
MSG = 16
HC = H * C

NC = 2
NS = 16
NW = NC * NS
EW = E // NW
G = 80
NCHUNK = EW // G
NPAD = 10240
ROWS_PER_TILE = NPAD // NS

BN = 2000
BE = 4000

KVW = 2 * HC
LW = 16
EXW = 8


def _sc_mesh():
  return plsc.VectorSubcoreMesh(
      core_axis_name="c", subcore_axis_name="s", num_cores=NC, num_subcores=NS)


def _node_proj_body(x_ref, lu_ref, wq_ref, bq_ref, wk_ref, bk_ref, wv_ref,
                    bv_ref, ws_ref, bs_ref, kv_ref, q_ref, skip_ref, lut_ref):
  x = x_ref[...]
  kv_ref[:, :HC] = jnp.dot(x, wk_ref[...], preferred_element_type=jnp.float32) + bk_ref[...]
  kv_ref[:, HC:] = jnp.dot(x, wv_ref[...], preferred_element_type=jnp.float32) + bv_ref[...]
  q_ref[...] = jnp.dot(x, wq_ref[...], preferred_element_type=jnp.float32) + bq_ref[...]
  skip_ref[...] = jnp.dot(x, ws_ref[...], preferred_element_type=jnp.float32) + bs_ref[...]
  lut_ref[...] = jnp.concatenate(
      [lu_ref[...], jnp.zeros((x.shape[0], LW - 1), jnp.float32)], axis=1)


def _node_proj(node_memory, last_update, Wq, bq, Wk, bk, Wv, bv, Ws, bs):
  wspec = pl.BlockSpec((D, HC), lambda i: (0, 0))
  bspec = pl.BlockSpec((1, HC), lambda i: (0, 0))
  return pl.pallas_call(
      _node_proj_body,
      grid=(N // BN,),
      in_specs=[pl.BlockSpec((BN, D), lambda i: (i, 0)),
                pl.BlockSpec((BN, 1), lambda i: (i, 0)),
                wspec, bspec, wspec, bspec, wspec, bspec, wspec, bspec],
      out_specs=[pl.BlockSpec((BN, KVW), lambda i: (i, 0)),
                 pl.BlockSpec((BN, HC), lambda i: (i, 0)),
                 pl.BlockSpec((BN, HC), lambda i: (i, 0)),
                 pl.BlockSpec((BN, LW), lambda i: (i, 0))],
      out_shape=[jax.ShapeDtypeStruct((N, KVW), jnp.float32),
                 jax.ShapeDtypeStruct((N, HC), jnp.float32),
                 jax.ShapeDtypeStruct((N, HC), jnp.float32),
                 jax.ShapeDtypeStruct((N, LW), jnp.float32)],
  )(node_memory, last_update[:, None], Wq, bq[None, :], Wk, bk[None, :],
    Wv, bv[None, :], Ws, bs[None, :])


def _gather_kernel(kv_tab, q_tab, lut_tab, src, dst):
  @functools.partial(
      pl.kernel,
      mesh=_sc_mesh(),
      out_type=(jax.ShapeDtypeStruct((E, KVW), jnp.float32),
                jax.ShapeDtypeStruct((E, HC), jnp.float32),
                jax.ShapeDtypeStruct((E, LW), jnp.float32)),
      scratch_types=[
          pltpu.VMEM((EW,), jnp.int32),
          pltpu.VMEM((EW,), jnp.int32),
          pltpu.VMEM((G, KVW), jnp.float32),
          pltpu.VMEM((G, HC), jnp.float32),
          pltpu.VMEM((G, LW), jnp.float32),
          pltpu.SemaphoreType.DMA,
          pltpu.SemaphoreType.DMA,
          pltpu.SemaphoreType.DMA,
      ],
      compiler_params=pltpu.CompilerParams(use_tc_tiling_on_sc=False),
  )
  def body(kv_hbm, q_hbm, lut_hbm, src_hbm, dst_hbm, kvj_hbm, qi_hbm, luj_hbm,
           src_v, dst_v, kv_v, q_v, lu_v, sem1, sem2, sem3):
    wid = lax.axis_index("s") * NC + lax.axis_index("c")
    base = wid * EW
    pltpu.sync_copy(src_hbm.at[pl.ds(base, EW)], src_v)
    pltpu.sync_copy(dst_hbm.at[pl.ds(base, EW)], dst_v)

    @pl.loop(0, NCHUNK)
    def _(j):
      o = base + j * G
      jo = j * G
      cp1 = pltpu.async_copy(kv_hbm.at[src_v.at[pl.ds(jo, G)]], kv_v, sem1)
      cp2 = pltpu.async_copy(q_hbm.at[dst_v.at[pl.ds(jo, G)]], q_v, sem2)
      cp3 = pltpu.async_copy(lut_hbm.at[src_v.at[pl.ds(jo, G)]], lu_v, sem3)
      cp1.wait()
      cp2.wait()
      cp3.wait()
      pltpu.sync_copy(kv_v, kvj_hbm.at[pl.ds(o, G)])
      pltpu.sync_copy(q_v, qi_hbm.at[pl.ds(o, G)])
      pltpu.sync_copy(lu_v, luj_hbm.at[pl.ds(o, G)])

  return body(kv_tab, q_tab, lut_tab, src, dst)


def _edge_proj_body(lut_ref, t_ref, msg_ref, wt_ref, bt_ref, wet_ref, wem_ref,
                    be_ref, e_ref):
  rel = jnp.maximum(t_ref[...] - lut_ref[:, :1], 0.0)
  enc = jnp.cos(rel * wt_ref[...] + bt_ref[...])
  e = jnp.dot(enc, wet_ref[...], preferred_element_type=jnp.float32)
  e += jnp.dot(msg_ref[...], wem_ref[...], preferred_element_type=jnp.float32)
  e_ref[...] = e + be_ref[...]


def _edge_proj(lut, event_t, event_msg, wt, bt, We, be):
  return pl.pallas_call(
      _edge_proj_body,
      grid=(E // BE,),
      in_specs=[pl.BlockSpec((BE, LW), lambda i: (i, 0)),
                pl.BlockSpec((BE, 1), lambda i: (i, 0)),
                pl.BlockSpec((BE, MSG), lambda i: (i, 0)),
                pl.BlockSpec((1, TD), lambda i: (0, 0)),
                pl.BlockSpec((1, TD), lambda i: (0, 0)),
                pl.BlockSpec((TD, HC), lambda i: (0, 0)),
                pl.BlockSpec((MSG, HC), lambda i: (0, 0)),
                pl.BlockSpec((1, HC), lambda i: (0, 0))],
      out_specs=pl.BlockSpec((BE, HC), lambda i: (i, 0)),
      out_shape=jax.ShapeDtypeStruct((E, HC), jnp.float32),
  )(lut, event_t[:, None], event_msg, wt[None, :], bt[None, :], We[:TD],
    We[TD:], be[None, :])


def _alpha_body(qi_ref, k_ref, e_ref, alpha_ref, amax_ref):
  p = qi_ref[...] * (k_ref[...] + e_ref[...])
  a0 = jnp.sum(p[:, :C], axis=1, keepdims=True)
  a1 = jnp.sum(p[:, C:], axis=1, keepdims=True)
  alpha = jnp.concatenate([a0, a1], axis=1) * (1.0 / 8.0)
  alpha_ref[...] = alpha
  m = jnp.max(alpha, axis=0, keepdims=True)

  @pl.when(pl.program_id(0) == 0)
  def _():
    amax_ref[...] = m

  @pl.when(pl.program_id(0) > 0)
  def _():
    amax_ref[...] = jnp.maximum(amax_ref[...], m)


def _alpha(qi, kvj, e_tab):
  return pl.pallas_call(
      _alpha_body,
      grid=(E // BE,),
      in_specs=[pl.BlockSpec((BE, HC), lambda i: (i, 0)),
                pl.BlockSpec((BE, HC), lambda i: (i, 0)),
                pl.BlockSpec((BE, HC), lambda i: (i, 0))],
      out_specs=[pl.BlockSpec((BE, H), lambda i: (i, 0)),
                 pl.BlockSpec((1, H), lambda i: (0, 0))],
      out_shape=[jax.ShapeDtypeStruct((E, H), jnp.float32),
                 jax.ShapeDtypeStruct((1, H), jnp.float32)],
  )(qi, kvj, e_tab)


def _wv_body(alpha_ref, amax_ref, v_ref, e_ref, w_ref, ex_ref):
  ex = jnp.exp(alpha_ref[...] - amax_ref[...])
  r = v_ref[...] + e_ref[...]
  w0 = r[:, :C] * ex[:, :1]
  w1 = r[:, C:] * ex[:, 1:2]
  w_ref[...] = jnp.concatenate([w0, w1], axis=1)
  ex_ref[...] = jnp.concatenate(
      [ex, jnp.zeros((ex.shape[0], EXW - H), jnp.float32)], axis=1)


def _wv(alpha, amax, kvj, e_tab):
  return pl.pallas_call(
      _wv_body,
      grid=(E // BE,),
      in_specs=[pl.BlockSpec((BE, H), lambda i: (i, 0)),
                pl.BlockSpec((1, H), lambda i: (0, 0)),
                pl.BlockSpec((BE, HC), lambda i: (i, 1)),
                pl.BlockSpec((BE, HC), lambda i: (i, 0))],
      out_specs=[pl.BlockSpec((BE, HC), lambda i: (i, 0)),
                 pl.BlockSpec((BE, EXW), lambda i: (i, 0))],
      out_shape=[jax.ShapeDtypeStruct((E, HC), jnp.float32),
                 jax.ShapeDtypeStruct((E, EXW), jnp.float32)],
  )(alpha, amax, kvj, e_tab)


def _scatter_kernel(w, ex8, dst, z128, z8):
  @functools.partial(
      pl.kernel,
      mesh=_sc_mesh(),
      out_type=(jax.ShapeDtypeStruct((NC * NPAD, HC), jnp.float32),
                jax.ShapeDtypeStruct((NC * NPAD, EXW), jnp.float32)),
      scratch_types=[
          pltpu.VMEM((EW,), jnp.int32),
          pltpu.VMEM((G, HC), jnp.float32),
          pltpu.VMEM((G, EXW), jnp.float32),
          pltpu.VMEM_SHARED((NPAD, HC), jnp.float32),
          pltpu.VMEM_SHARED((NPAD, EXW), jnp.float32),
      ],
      compiler_params=pltpu.CompilerParams(use_tc_tiling_on_sc=False),
  )
  def body(w_hbm, ex_hbm, dst_hbm, z128_hbm, z8_hbm, nump_hbm, denp_hbm,
           dst_v, w_v, ex_v, acc_sh, den_sh):
    cid = lax.axis_index("c")
    sid = lax.axis_index("s")
    wid = sid * NC + cid
    base = wid * EW
    rbase = sid * ROWS_PER_TILE

    pltpu.sync_copy(dst_hbm.at[pl.ds(base, EW)], dst_v)

    pltpu.sync_copy(z128_hbm.at[pl.ds(rbase, ROWS_PER_TILE)],
                    acc_sh.at[pl.ds(rbase, ROWS_PER_TILE)])
    pltpu.sync_copy(z8_hbm.at[pl.ds(rbase, ROWS_PER_TILE)],
                    den_sh.at[pl.ds(rbase, ROWS_PER_TILE)])

    plsc.subcore_barrier()

    @pl.loop(0, NCHUNK)
    def _(j):
      o = base + j * G
      pltpu.sync_copy(w_hbm.at[pl.ds(o, G)], w_v)
      pltpu.sync_copy(ex_hbm.at[pl.ds(o, G)], ex_v)
      pltpu.sync_copy(w_v, acc_sh.at[dst_v.at[pl.ds(j * G, G)]], add=True)
      pltpu.sync_copy(ex_v, den_sh.at[dst_v.at[pl.ds(j * G, G)]], add=True)

    plsc.subcore_barrier()

    obase = cid * NPAD + rbase
    pltpu.sync_copy(acc_sh.at[pl.ds(rbase, ROWS_PER_TILE)],
                    nump_hbm.at[pl.ds(obase, ROWS_PER_TILE)])
    pltpu.sync_copy(den_sh.at[pl.ds(rbase, ROWS_PER_TILE)],
                    denp_hbm.at[pl.ds(obase, ROWS_PER_TILE)])

  return body(w, ex8, dst, z128, z8)


def _final_body(nump_ref, denp_ref, skip_ref, out_ref):
  num = nump_ref[0] + nump_ref[1]
  den = denp_ref[0, :, :H] + denp_ref[1, :, :H]
  d0 = jnp.broadcast_to(den[:, :1], (num.shape[0], C))
  d1 = jnp.broadcast_to(den[:, 1:2], (num.shape[0], C))
  dfull = jnp.concatenate([d0, d1], axis=1)
  out_ref[...] = num / (dfull + 1e-30) + skip_ref[...]


def _finalize(nump, denp, skip):
  return pl.pallas_call(
      _final_body,
      grid=(N // BN,),
      in_specs=[pl.BlockSpec((NC, BN, HC), lambda i: (0, i, 0)),
                pl.BlockSpec((NC, BN, EXW), lambda i: (0, i, 0)),
                pl.BlockSpec((BN, HC), lambda i: (i, 0))],
      out_specs=pl.BlockSpec((BN, HC), lambda i: (i, 0)),
      out_shape=jax.ShapeDtypeStruct((N, HC), jnp.float32),
  )(nump, denp, skip)


def kernel(node_memory, last_update, edge_index, event_t, event_msg,
           wt, bt, Wq, bq, Wk, bk, Wv, bv, We, be, Ws, bs):
  src = edge_index[0]
  dst = edge_index[1]
  kv_tab, q_tab, skip, lut_tab = _node_proj(node_memory, last_update,
                                            Wq, bq, Wk, bk, Wv, bv, Ws, bs)
  kvj, qi, luj = _gather_kernel(kv_tab, q_tab, lut_tab, src, dst)
  e_tab = _edge_proj(luj, event_t, event_msg, wt, bt, We, be)
  alpha, amax = _alpha(qi, kvj, e_tab)
  w, ex8 = _wv(alpha, amax, kvj, e_tab)
  z128 = jnp.zeros((NPAD, HC), jnp.float32)
  z8 = jnp.zeros((NPAD, EXW), jnp.float32)
  nump, denp = _scatter_kernel(w, ex8, dst, z128, z8)
  nump = nump.reshape(NC, NPAD, HC)
  denp = denp.reshape(NC, NPAD, EXW)
  return _finalize(nump, denp, skip)


# speedup vs baseline: 12.8685x; 1.0109x over previous
import functools

import jax
import jax.numpy as jnp
from jax import lax
from jax.experimental import pallas as pl
from jax.experimental.pallas import tpu as pltpu
from jax.experimental.pallas import tpu_sc as plsc

N = 10000
E = 320000
D = 128
H = 2
C = 64
TD = 32
MSG = 16
HC = H * C

NC = 2
NS = 16
NW = NC * NS
EW = E // NW
G = 80
NCHUNK = EW // G
NPAD = 10240
ROWS_PER_TILE = NPAD // NS

BN = 2000
BE = 4000

KVW = 2 * HC
LW = 16
EXW = 8


def _sc_mesh():
  return plsc.VectorSubcoreMesh(
      core_axis_name="c", subcore_axis_name="s", num_cores=NC, num_subcores=NS)


def _node_proj_body(x_ref, lu_ref, wq_ref, bq_ref, wk_ref, bk_ref, wv_ref,
                    bv_ref, ws_ref, bs_ref, kv_ref, q_ref, skip_ref, lut_ref):
  x = x_ref[...]
  kv_ref[:, :HC] = jnp.dot(x, wk_ref[...], preferred_element_type=jnp.float32) + bk_ref[...]
  kv_ref[:, HC:] = jnp.dot(x, wv_ref[...], preferred_element_type=jnp.float32) + bv_ref[...]
  q_ref[...] = jnp.dot(x, wq_ref[...], preferred_element_type=jnp.float32) + bq_ref[...]
  skip_ref[...] = jnp.dot(x, ws_ref[...], preferred_element_type=jnp.float32) + bs_ref[...]
  lut_ref[...] = jnp.concatenate(
      [lu_ref[...], jnp.zeros((x.shape[0], LW - 1), jnp.float32)], axis=1)


def _node_proj(node_memory, last_update, Wq, bq, Wk, bk, Wv, bv, Ws, bs):
  wspec = pl.BlockSpec((D, HC), lambda i: (0, 0))
  bspec = pl.BlockSpec((1, HC), lambda i: (0, 0))
  return pl.pallas_call(
      _node_proj_body,
      grid=(N // BN,),
      in_specs=[pl.BlockSpec((BN, D), lambda i: (i, 0)),
                pl.BlockSpec((BN, 1), lambda i: (i, 0)),
                wspec, bspec, wspec, bspec, wspec, bspec, wspec, bspec],
      out_specs=[pl.BlockSpec((BN, KVW), lambda i: (i, 0)),
                 pl.BlockSpec((BN, HC), lambda i: (i, 0)),
                 pl.BlockSpec((BN, HC), lambda i: (i, 0)),
                 pl.BlockSpec((BN, LW), lambda i: (i, 0))],
      out_shape=[jax.ShapeDtypeStruct((N, KVW), jnp.float32),
                 jax.ShapeDtypeStruct((N, HC), jnp.float32),
                 jax.ShapeDtypeStruct((N, HC), jnp.float32),
                 jax.ShapeDtypeStruct((N, LW), jnp.float32)],
  )(node_memory, last_update[:, None], Wq, bq[None, :], Wk, bk[None, :],
    Wv, bv[None, :], Ws, bs[None, :])


def _gather_kernel(kv_tab, q_tab, lut_tab, src, dst):
  @functools.partial(
      pl.kernel,
      mesh=_sc_mesh(),
      out_type=(jax.ShapeDtypeStruct((E, KVW), jnp.float32),
                jax.ShapeDtypeStruct((E, HC), jnp.float32),
                jax.ShapeDtypeStruct((E, LW), jnp.float32)),
      scratch_types=[
          pltpu.VMEM((EW,), jnp.int32),
          pltpu.VMEM((EW,), jnp.int32),
          pltpu.VMEM((2, G, KVW), jnp.float32),
          pltpu.VMEM((2, G, HC), jnp.float32),
          pltpu.VMEM((2, G, LW), jnp.float32),
          pltpu.SemaphoreType.DMA((2,)),
          pltpu.SemaphoreType.DMA((2,)),
          pltpu.SemaphoreType.DMA((2,)),
          pltpu.SemaphoreType.DMA((2,)),
          pltpu.SemaphoreType.DMA((2,)),
          pltpu.SemaphoreType.DMA((2,)),
      ],
      compiler_params=pltpu.CompilerParams(use_tc_tiling_on_sc=False),
  )
  def body(kv_hbm, q_hbm, lut_hbm, src_hbm, dst_hbm, kvj_hbm, qi_hbm, luj_hbm,
           src_v, dst_v, kv_v, q_v, lu_v, gkv, gq, glu, wkv, wq, wlu):
    wid = lax.axis_index("s") * NC + lax.axis_index("c")
    base = wid * EW
    pltpu.sync_copy(src_hbm.at[pl.ds(base, EW)], src_v)
    pltpu.sync_copy(dst_hbm.at[pl.ds(base, EW)], dst_v)

    def start_gather(j, b):
      jo = j * G
      pltpu.async_copy(kv_hbm.at[src_v.at[pl.ds(jo, G)]], kv_v.at[b], gkv.at[b])
      pltpu.async_copy(q_hbm.at[dst_v.at[pl.ds(jo, G)]], q_v.at[b], gq.at[b])
      pltpu.async_copy(lut_hbm.at[src_v.at[pl.ds(jo, G)]], lu_v.at[b], glu.at[b])

    def wait_gather(b):
      pltpu.make_async_copy(kv_hbm.at[pl.ds(0, G)], kv_v.at[b], gkv.at[b]).wait()
      pltpu.make_async_copy(q_hbm.at[pl.ds(0, G)], q_v.at[b], gq.at[b]).wait()
      pltpu.make_async_copy(lut_hbm.at[pl.ds(0, G)], lu_v.at[b], glu.at[b]).wait()

    def start_wb(j, b):
      o = base + j * G
      pltpu.async_copy(kv_v.at[b], kvj_hbm.at[pl.ds(o, G)], wkv.at[b])
      pltpu.async_copy(q_v.at[b], qi_hbm.at[pl.ds(o, G)], wq.at[b])
      pltpu.async_copy(lu_v.at[b], luj_hbm.at[pl.ds(o, G)], wlu.at[b])

    def wait_wb(b):
      pltpu.make_async_copy(kv_v.at[b], kvj_hbm.at[pl.ds(0, G)], wkv.at[b]).wait()
      pltpu.make_async_copy(q_v.at[b], qi_hbm.at[pl.ds(0, G)], wq.at[b]).wait()
      pltpu.make_async_copy(lu_v.at[b], luj_hbm.at[pl.ds(0, G)], wlu.at[b]).wait()

    start_gather(0, 0)
    wait_gather(0)
    start_wb(0, 0)
    start_gather(1, 1)

    @pl.loop(0, (NCHUNK - 3) // 2)
    def _(g):
      j = 1 + 2 * g
      for b, dj in ((1, 0), (0, 1)):
        wait_gather(b)
        start_wb(j + dj, b)
        wait_wb(1 - b)
        start_gather(j + dj + 1, 1 - b)

    wait_gather(1)
    start_wb(NCHUNK - 2, 1)
    wait_wb(0)
    start_gather(NCHUNK - 1, 0)
    wait_gather(0)
    start_wb(NCHUNK - 1, 0)
    wait_wb(1)
    wait_wb(0)

  return body(kv_tab, q_tab, lut_tab, src, dst)


def _edge_proj_body(lut_ref, t_ref, msg_ref, wt_ref, bt_ref, wet_ref, wem_ref,
                    be_ref, e_ref):
  rel = jnp.maximum(t_ref[...] - lut_ref[:, :1], 0.0)
  enc = jnp.cos(rel * wt_ref[...] + bt_ref[...])
  e = jnp.dot(enc, wet_ref[...], preferred_element_type=jnp.float32)
  e += jnp.dot(msg_ref[...], wem_ref[...], preferred_element_type=jnp.float32)
  e_ref[...] = e + be_ref[...]


def _edge_proj(lut, event_t, event_msg, wt, bt, We, be):
  return pl.pallas_call(
      _edge_proj_body,
      grid=(E // BE,),
      in_specs=[pl.BlockSpec((BE, LW), lambda i: (i, 0)),
                pl.BlockSpec((BE, 1), lambda i: (i, 0)),
                pl.BlockSpec((BE, MSG), lambda i: (i, 0)),
                pl.BlockSpec((1, TD), lambda i: (0, 0)),
                pl.BlockSpec((1, TD), lambda i: (0, 0)),
                pl.BlockSpec((TD, HC), lambda i: (0, 0)),
                pl.BlockSpec((MSG, HC), lambda i: (0, 0)),
                pl.BlockSpec((1, HC), lambda i: (0, 0))],
      out_specs=pl.BlockSpec((BE, HC), lambda i: (i, 0)),
      out_shape=jax.ShapeDtypeStruct((E, HC), jnp.float32),
  )(lut, event_t[:, None], event_msg, wt[None, :], bt[None, :], We[:TD],
    We[TD:], be[None, :])


def _alpha_body(qi_ref, k_ref, e_ref, alpha_ref, amax_ref):
  p = qi_ref[...] * (k_ref[...] + e_ref[...])
  a0 = jnp.sum(p[:, :C], axis=1, keepdims=True)
  a1 = jnp.sum(p[:, C:], axis=1, keepdims=True)
  alpha = jnp.concatenate([a0, a1], axis=1) * (1.0 / 8.0)
  alpha_ref[...] = alpha
  m = jnp.max(alpha, axis=0, keepdims=True)

  @pl.when(pl.program_id(0) == 0)
  def _():
    amax_ref[...] = m

  @pl.when(pl.program_id(0) > 0)
  def _():
    amax_ref[...] = jnp.maximum(amax_ref[...], m)


def _alpha(qi, kvj, e_tab):
  return pl.pallas_call(
      _alpha_body,
      grid=(E // BE,),
      in_specs=[pl.BlockSpec((BE, HC), lambda i: (i, 0)),
                pl.BlockSpec((BE, HC), lambda i: (i, 0)),
                pl.BlockSpec((BE, HC), lambda i: (i, 0))],
      out_specs=[pl.BlockSpec((BE, H), lambda i: (i, 0)),
                 pl.BlockSpec((1, H), lambda i: (0, 0))],
      out_shape=[jax.ShapeDtypeStruct((E, H), jnp.float32),
                 jax.ShapeDtypeStruct((1, H), jnp.float32)],
  )(qi, kvj, e_tab)


def _wv_body(alpha_ref, amax_ref, v_ref, e_ref, w_ref, ex_ref):
  ex = jnp.exp(alpha_ref[...] - amax_ref[...])
  r = v_ref[...] + e_ref[...]
  w0 = r[:, :C] * ex[:, :1]
  w1 = r[:, C:] * ex[:, 1:2]
  w_ref[...] = jnp.concatenate([w0, w1], axis=1)
  ex_ref[...] = jnp.concatenate(
      [ex, jnp.zeros((ex.shape[0], EXW - H), jnp.float32)], axis=1)


def _wv(alpha, amax, kvj, e_tab):
  return pl.pallas_call(
      _wv_body,
      grid=(E // BE,),
      in_specs=[pl.BlockSpec((BE, H), lambda i: (i, 0)),
                pl.BlockSpec((1, H), lambda i: (0, 0)),
                pl.BlockSpec((BE, HC), lambda i: (i, 1)),
                pl.BlockSpec((BE, HC), lambda i: (i, 0))],
      out_specs=[pl.BlockSpec((BE, HC), lambda i: (i, 0)),
                 pl.BlockSpec((BE, EXW), lambda i: (i, 0))],
      out_shape=[jax.ShapeDtypeStruct((E, HC), jnp.float32),
                 jax.ShapeDtypeStruct((E, EXW), jnp.float32)],
  )(alpha, amax, kvj, e_tab)


def _scatter_kernel(w, ex8, dst, z128, z8):
  @functools.partial(
      pl.kernel,
      mesh=_sc_mesh(),
      out_type=(jax.ShapeDtypeStruct((NC * NPAD, HC), jnp.float32),
                jax.ShapeDtypeStruct((NC * NPAD, EXW), jnp.float32)),
      scratch_types=[
          pltpu.VMEM((EW,), jnp.int32),
          pltpu.VMEM((G, HC), jnp.float32),
          pltpu.VMEM((G, EXW), jnp.float32),
          pltpu.VMEM_SHARED((NPAD, HC), jnp.float32),
          pltpu.VMEM_SHARED((NPAD, EXW), jnp.float32),
      ],
      compiler_params=pltpu.CompilerParams(use_tc_tiling_on_sc=False),
  )
  def body(w_hbm, ex_hbm, dst_hbm, z128_hbm, z8_hbm, nump_hbm, denp_hbm,
           dst_v, w_v, ex_v, acc_sh, den_sh):
    cid = lax.axis_index("c")
    sid = lax.axis_index("s")
    wid = sid * NC + cid
    base = wid * EW
    rbase = sid * ROWS_PER_TILE

    pltpu.sync_copy(dst_hbm.at[pl.ds(base, EW)], dst_v)

    pltpu.sync_copy(z128_hbm.at[pl.ds(rbase, ROWS_PER_TILE)],
                    acc_sh.at[pl.ds(rbase, ROWS_PER_TILE)])
    pltpu.sync_copy(z8_hbm.at[pl.ds(rbase, ROWS_PER_TILE)],
                    den_sh.at[pl.ds(rbase, ROWS_PER_TILE)])

    plsc.subcore_barrier()

    @pl.loop(0, NCHUNK)
    def _(j):
      o = base + j * G
      pltpu.sync_copy(w_hbm.at[pl.ds(o, G)], w_v)
      pltpu.sync_copy(ex_hbm.at[pl.ds(o, G)], ex_v)
      pltpu.sync_copy(w_v, acc_sh.at[dst_v.at[pl.ds(j * G, G)]], add=True)
      pltpu.sync_copy(ex_v, den_sh.at[dst_v.at[pl.ds(j * G, G)]], add=True)

    plsc.subcore_barrier()

    obase = cid * NPAD + rbase
    pltpu.sync_copy(acc_sh.at[pl.ds(rbase, ROWS_PER_TILE)],
                    nump_hbm.at[pl.ds(obase, ROWS_PER_TILE)])
    pltpu.sync_copy(den_sh.at[pl.ds(rbase, ROWS_PER_TILE)],
                    denp_hbm.at[pl.ds(obase, ROWS_PER_TILE)])

  return body(w, ex8, dst, z128, z8)


def _final_body(nump_ref, denp_ref, skip_ref, out_ref):
  num = nump_ref[0] + nump_ref[1]
  den = denp_ref[0, :, :H] + denp_ref[1, :, :H]
  d0 = jnp.broadcast_to(den[:, :1], (num.shape[0], C))
  d1 = jnp.broadcast_to(den[:, 1:2], (num.shape[0], C))
  dfull = jnp.concatenate([d0, d1], axis=1)
  out_ref[...] = num / (dfull + 1e-30) + skip_ref[...]


def _finalize(nump, denp, skip):
  return pl.pallas_call(
      _final_body,
      grid=(N // BN,),
      in_specs=[pl.BlockSpec((NC, BN, HC), lambda i: (0, i, 0)),
                pl.BlockSpec((NC, BN, EXW), lambda i: (0, i, 0)),
                pl.BlockSpec((BN, HC), lambda i: (i, 0))],
      out_specs=pl.BlockSpec((BN, HC), lambda i: (i, 0)),
      out_shape=jax.ShapeDtypeStruct((N, HC), jnp.float32),
  )(nump, denp, skip)


def kernel(node_memory, last_update, edge_index, event_t, event_msg,
           wt, bt, Wq, bq, Wk, bk, Wv, bv, We, be, Ws, bs):
  src = edge_index[0]
  dst = edge_index[1]
  kv_tab, q_tab, skip, lut_tab = _node_proj(node_memory, last_update,
                                            Wq, bq, Wk, bk, Wv, bv, Ws, bs)
  kvj, qi, luj = _gather_kernel(kv_tab, q_tab, lut_tab, src, dst)
  e_tab = _edge_proj(luj, event_t, event_msg, wt, bt, We, be)
  alpha, amax = _alpha(qi, kvj, e_tab)
  w, ex8 = _wv(alpha, amax, kvj, e_tab)
  z128 = jnp.zeros((NPAD, HC), jnp.float32)
  z8 = jnp.zeros((NPAD, EXW), jnp.float32)
  nump, denp = _scatter_kernel(w, ex8, dst, z128, z8)
  nump = nump.reshape(NC, NPAD, HC)
  denp = denp.reshape(NC, NPAD, EXW)
  return _finalize(nump, denp, skip)


# speedup vs baseline: 13.3944x; 1.0409x over previous
import functools

import jax
import jax.numpy as jnp
from jax import lax
from jax.experimental import pallas as pl
from jax.experimental.pallas import tpu as pltpu
from jax.experimental.pallas import tpu_sc as plsc

N = 10000
E = 320000
D = 128
H = 2
C = 64
TD = 32
MSG = 16
HC = H * C

NC = 2
NS = 16
NW = NC * NS
EW = E // NW
G = 80
NCHUNK = EW // G
NPAD = 10240
ROWS_PER_TILE = NPAD // NS

BN = 2000
BE = 4000

KVW = 2 * HC
LW = 16
EXW = 8


def _sc_mesh():
  return plsc.VectorSubcoreMesh(
      core_axis_name="c", subcore_axis_name="s", num_cores=NC, num_subcores=NS)


def _node_proj_body(x_ref, lu_ref, wq_ref, bq_ref, wk_ref, bk_ref, wv_ref,
                    bv_ref, ws_ref, bs_ref, kv_ref, q_ref, skip_ref, lut_ref):
  x = x_ref[...]
  kv_ref[:, :HC] = jnp.dot(x, wk_ref[...], preferred_element_type=jnp.float32) + bk_ref[...]
  kv_ref[:, HC:] = jnp.dot(x, wv_ref[...], preferred_element_type=jnp.float32) + bv_ref[...]
  q_ref[...] = jnp.dot(x, wq_ref[...], preferred_element_type=jnp.float32) + bq_ref[...]
  skip_ref[...] = jnp.dot(x, ws_ref[...], preferred_element_type=jnp.float32) + bs_ref[...]
  lut_ref[...] = jnp.concatenate(
      [lu_ref[...], jnp.zeros((x.shape[0], LW - 1), jnp.float32)], axis=1)


def _node_proj(node_memory, last_update, Wq, bq, Wk, bk, Wv, bv, Ws, bs):
  wspec = pl.BlockSpec((D, HC), lambda i: (0, 0))
  bspec = pl.BlockSpec((1, HC), lambda i: (0, 0))
  return pl.pallas_call(
      _node_proj_body,
      grid=(N // BN,),
      in_specs=[pl.BlockSpec((BN, D), lambda i: (i, 0)),
                pl.BlockSpec((BN, 1), lambda i: (i, 0)),
                wspec, bspec, wspec, bspec, wspec, bspec, wspec, bspec],
      out_specs=[pl.BlockSpec((BN, KVW), lambda i: (i, 0)),
                 pl.BlockSpec((BN, HC), lambda i: (i, 0)),
                 pl.BlockSpec((BN, HC), lambda i: (i, 0)),
                 pl.BlockSpec((BN, LW), lambda i: (i, 0))],
      out_shape=[jax.ShapeDtypeStruct((N, KVW), jnp.float32),
                 jax.ShapeDtypeStruct((N, HC), jnp.float32),
                 jax.ShapeDtypeStruct((N, HC), jnp.float32),
                 jax.ShapeDtypeStruct((N, LW), jnp.float32)],
  )(node_memory, last_update[:, None], Wq, bq[None, :], Wk, bk[None, :],
    Wv, bv[None, :], Ws, bs[None, :])


def _gather_kernel(kv_tab, q_tab, lut_tab, src, dst):
  @functools.partial(
      pl.kernel,
      mesh=_sc_mesh(),
      out_type=(jax.ShapeDtypeStruct((E, KVW), jnp.float32),
                jax.ShapeDtypeStruct((E, HC), jnp.float32),
                jax.ShapeDtypeStruct((E, LW), jnp.float32)),
      scratch_types=[
          pltpu.VMEM((EW,), jnp.int32),
          pltpu.VMEM((EW,), jnp.int32),
          pltpu.VMEM((2, G, KVW), jnp.float32),
          pltpu.VMEM((2, G, HC), jnp.float32),
          pltpu.VMEM((2, G, LW), jnp.float32),
          pltpu.SemaphoreType.DMA((2,)),
          pltpu.SemaphoreType.DMA((2,)),
          pltpu.SemaphoreType.DMA((2,)),
          pltpu.SemaphoreType.DMA((2,)),
          pltpu.SemaphoreType.DMA((2,)),
          pltpu.SemaphoreType.DMA((2,)),
      ],
      compiler_params=pltpu.CompilerParams(use_tc_tiling_on_sc=False),
  )
  def body(kv_hbm, q_hbm, lut_hbm, src_hbm, dst_hbm, kvj_hbm, qi_hbm, luj_hbm,
           src_v, dst_v, kv_v, q_v, lu_v, gkv, gq, glu, wkv, wq, wlu):
    wid = lax.axis_index("s") * NC + lax.axis_index("c")
    base = wid * EW
    pltpu.sync_copy(src_hbm.at[pl.ds(base, EW)], src_v)
    pltpu.sync_copy(dst_hbm.at[pl.ds(base, EW)], dst_v)

    def start_gather(j, b):
      jo = j * G
      pltpu.async_copy(kv_hbm.at[src_v.at[pl.ds(jo, G)]], kv_v.at[b], gkv.at[b])
      pltpu.async_copy(q_hbm.at[dst_v.at[pl.ds(jo, G)]], q_v.at[b], gq.at[b])
      pltpu.async_copy(lut_hbm.at[src_v.at[pl.ds(jo, G)]], lu_v.at[b], glu.at[b])

    def wait_gather(b):
      pltpu.make_async_copy(kv_hbm.at[pl.ds(0, G)], kv_v.at[b], gkv.at[b]).wait()
      pltpu.make_async_copy(q_hbm.at[pl.ds(0, G)], q_v.at[b], gq.at[b]).wait()
      pltpu.make_async_copy(lut_hbm.at[pl.ds(0, G)], lu_v.at[b], glu.at[b]).wait()

    def start_wb(j, b):
      o = base + j * G
      pltpu.async_copy(kv_v.at[b], kvj_hbm.at[pl.ds(o, G)], wkv.at[b])
      pltpu.async_copy(q_v.at[b], qi_hbm.at[pl.ds(o, G)], wq.at[b])
      pltpu.async_copy(lu_v.at[b], luj_hbm.at[pl.ds(o, G)], wlu.at[b])

    def wait_wb(b):
      pltpu.make_async_copy(kv_v.at[b], kvj_hbm.at[pl.ds(0, G)], wkv.at[b]).wait()
      pltpu.make_async_copy(q_v.at[b], qi_hbm.at[pl.ds(0, G)], wq.at[b]).wait()
      pltpu.make_async_copy(lu_v.at[b], luj_hbm.at[pl.ds(0, G)], wlu.at[b]).wait()

    start_gather(0, 0)
    wait_gather(0)
    start_wb(0, 0)
    start_gather(1, 1)

    @pl.loop(0, (NCHUNK - 3) // 2)
    def _(g):
      j = 1 + 2 * g
      for b, dj in ((1, 0), (0, 1)):
        wait_gather(b)
        start_wb(j + dj, b)
        wait_wb(1 - b)
        start_gather(j + dj + 1, 1 - b)

    wait_gather(1)
    start_wb(NCHUNK - 2, 1)
    wait_wb(0)
    start_gather(NCHUNK - 1, 0)
    wait_gather(0)
    start_wb(NCHUNK - 1, 0)
    wait_wb(1)
    wait_wb(0)

  return body(kv_tab, q_tab, lut_tab, src, dst)


def _alpha_e_body(lut_ref, t_ref, msg_ref, wt_ref, bt_ref, wet_ref, wem_ref,
                  be_ref, qi_ref, k_ref, e_ref, alpha_ref, amax_ref):
  rel = jnp.maximum(t_ref[...] - lut_ref[:, :1], 0.0)
  enc = jnp.cos(rel * wt_ref[...] + bt_ref[...])
  e = jnp.dot(enc, wet_ref[...], preferred_element_type=jnp.float32)
  e += jnp.dot(msg_ref[...], wem_ref[...], preferred_element_type=jnp.float32)
  e += be_ref[...]
  e_ref[...] = e
  p = qi_ref[...] * (k_ref[...] + e)
  a0 = jnp.sum(p[:, :C], axis=1, keepdims=True)
  a1 = jnp.sum(p[:, C:], axis=1, keepdims=True)
  alpha = jnp.concatenate([a0, a1], axis=1) * (1.0 / 8.0)
  alpha_ref[...] = alpha
  m = jnp.max(alpha, axis=0, keepdims=True)

  @pl.when(pl.program_id(0) == 0)
  def _():
    amax_ref[...] = m

  @pl.when(pl.program_id(0) > 0)
  def _():
    amax_ref[...] = jnp.maximum(amax_ref[...], m)


def _alpha_e(lut, event_t, event_msg, wt, bt, We, be, qi, kvj):
  return pl.pallas_call(
      _alpha_e_body,
      grid=(E // BE,),
      in_specs=[pl.BlockSpec((BE, LW), lambda i: (i, 0)),
                pl.BlockSpec((BE, 1), lambda i: (i, 0)),
                pl.BlockSpec((BE, MSG), lambda i: (i, 0)),
                pl.BlockSpec((1, TD), lambda i: (0, 0)),
                pl.BlockSpec((1, TD), lambda i: (0, 0)),
                pl.BlockSpec((TD, HC), lambda i: (0, 0)),
                pl.BlockSpec((MSG, HC), lambda i: (0, 0)),
                pl.BlockSpec((1, HC), lambda i: (0, 0)),
                pl.BlockSpec((BE, HC), lambda i: (i, 0)),
                pl.BlockSpec((BE, HC), lambda i: (i, 0))],
      out_specs=[pl.BlockSpec((BE, HC), lambda i: (i, 0)),
                 pl.BlockSpec((BE, H), lambda i: (i, 0)),
                 pl.BlockSpec((1, H), lambda i: (0, 0))],
      out_shape=[jax.ShapeDtypeStruct((E, HC), jnp.float32),
                 jax.ShapeDtypeStruct((E, H), jnp.float32),
                 jax.ShapeDtypeStruct((1, H), jnp.float32)],
  )(lut, event_t[:, None], event_msg, wt[None, :], bt[None, :], We[:TD],
    We[TD:], be[None, :], qi, kvj)


def _wv_body(alpha_ref, amax_ref, v_ref, e_ref, w_ref, ex_ref):
  ex = jnp.exp(alpha_ref[...] - amax_ref[...])
  r = v_ref[...] + e_ref[...]
  w0 = r[:, :C] * ex[:, :1]
  w1 = r[:, C:] * ex[:, 1:2]
  w_ref[...] = jnp.concatenate([w0, w1], axis=1)
  ex_ref[...] = jnp.concatenate(
      [ex, jnp.zeros((ex.shape[0], EXW - H), jnp.float32)], axis=1)


def _wv(alpha, amax, kvj, e_tab):
  return pl.pallas_call(
      _wv_body,
      grid=(E // BE,),
      in_specs=[pl.BlockSpec((BE, H), lambda i: (i, 0)),
                pl.BlockSpec((1, H), lambda i: (0, 0)),
                pl.BlockSpec((BE, HC), lambda i: (i, 1)),
                pl.BlockSpec((BE, HC), lambda i: (i, 0))],
      out_specs=[pl.BlockSpec((BE, HC), lambda i: (i, 0)),
                 pl.BlockSpec((BE, EXW), lambda i: (i, 0))],
      out_shape=[jax.ShapeDtypeStruct((E, HC), jnp.float32),
                 jax.ShapeDtypeStruct((E, EXW), jnp.float32)],
  )(alpha, amax, kvj, e_tab)


def _scatter_kernel(w, ex8, dst, z128, z8):
  @functools.partial(
      pl.kernel,
      mesh=_sc_mesh(),
      out_type=(jax.ShapeDtypeStruct((NC * NPAD, HC), jnp.float32),
                jax.ShapeDtypeStruct((NC * NPAD, EXW), jnp.float32)),
      scratch_types=[
          pltpu.VMEM((EW,), jnp.int32),
          pltpu.VMEM((G, HC), jnp.float32),
          pltpu.VMEM((G, EXW), jnp.float32),
          pltpu.VMEM_SHARED((NPAD, HC), jnp.float32),
          pltpu.VMEM_SHARED((NPAD, EXW), jnp.float32),
      ],
      compiler_params=pltpu.CompilerParams(use_tc_tiling_on_sc=False),
  )
  def body(w_hbm, ex_hbm, dst_hbm, z128_hbm, z8_hbm, nump_hbm, denp_hbm,
           dst_v, w_v, ex_v, acc_sh, den_sh):
    cid = lax.axis_index("c")
    sid = lax.axis_index("s")
    wid = sid * NC + cid
    base = wid * EW
    rbase = sid * ROWS_PER_TILE

    pltpu.sync_copy(dst_hbm.at[pl.ds(base, EW)], dst_v)

    pltpu.sync_copy(z128_hbm.at[pl.ds(rbase, ROWS_PER_TILE)],
                    acc_sh.at[pl.ds(rbase, ROWS_PER_TILE)])
    pltpu.sync_copy(z8_hbm.at[pl.ds(rbase, ROWS_PER_TILE)],
                    den_sh.at[pl.ds(rbase, ROWS_PER_TILE)])

    plsc.subcore_barrier()

    @pl.loop(0, NCHUNK)
    def _(j):
      o = base + j * G
      pltpu.sync_copy(w_hbm.at[pl.ds(o, G)], w_v)
      pltpu.sync_copy(ex_hbm.at[pl.ds(o, G)], ex_v)
      pltpu.sync_copy(w_v, acc_sh.at[dst_v.at[pl.ds(j * G, G)]], add=True)
      pltpu.sync_copy(ex_v, den_sh.at[dst_v.at[pl.ds(j * G, G)]], add=True)

    plsc.subcore_barrier()

    obase = cid * NPAD + rbase
    pltpu.sync_copy(acc_sh.at[pl.ds(rbase, ROWS_PER_TILE)],
                    nump_hbm.at[pl.ds(obase, ROWS_PER_TILE)])
    pltpu.sync_copy(den_sh.at[pl.ds(rbase, ROWS_PER_TILE)],
                    denp_hbm.at[pl.ds(obase, ROWS_PER_TILE)])

  return body(w, ex8, dst, z128, z8)


def _final_body(nump_ref, denp_ref, skip_ref, out_ref):
  num = nump_ref[0] + nump_ref[1]
  den = denp_ref[0, :, :H] + denp_ref[1, :, :H]
  d0 = jnp.broadcast_to(den[:, :1], (num.shape[0], C))
  d1 = jnp.broadcast_to(den[:, 1:2], (num.shape[0], C))
  dfull = jnp.concatenate([d0, d1], axis=1)
  out_ref[...] = num / (dfull + 1e-30) + skip_ref[...]


def _finalize(nump, denp, skip):
  return pl.pallas_call(
      _final_body,
      grid=(N // BN,),
      in_specs=[pl.BlockSpec((NC, BN, HC), lambda i: (0, i, 0)),
                pl.BlockSpec((NC, BN, EXW), lambda i: (0, i, 0)),
                pl.BlockSpec((BN, HC), lambda i: (i, 0))],
      out_specs=pl.BlockSpec((BN, HC), lambda i: (i, 0)),
      out_shape=jax.ShapeDtypeStruct((N, HC), jnp.float32),
  )(nump, denp, skip)


def kernel(node_memory, last_update, edge_index, event_t, event_msg,
           wt, bt, Wq, bq, Wk, bk, Wv, bv, We, be, Ws, bs):
  src = edge_index[0]
  dst = edge_index[1]
  kv_tab, q_tab, skip, lut_tab = _node_proj(node_memory, last_update,
                                            Wq, bq, Wk, bk, Wv, bv, Ws, bs)
  kvj, qi, luj = _gather_kernel(kv_tab, q_tab, lut_tab, src, dst)
  e_tab, alpha, amax = _alpha_e(luj, event_t, event_msg, wt, bt, We, be,
                                qi, kvj)
  w, ex8 = _wv(alpha, amax, kvj, e_tab)
  z128 = jnp.zeros((NPAD, HC), jnp.float32)
  z8 = jnp.zeros((NPAD, EXW), jnp.float32)
  nump, denp = _scatter_kernel(w, ex8, dst, z128, z8)
  nump = nump.reshape(NC, NPAD, HC)
  denp = denp.reshape(NC, NPAD, EXW)
  return _finalize(nump, denp, skip)
